# baseline (speedup 1.0000x reference)
import jax
import jax.numpy as jnp
from jax.experimental import pallas as pl

_ENT_DIM = 2048
_EXPANSION = 16
_K = 512
_IN_DIM = _ENT_DIM + 1
_OUT_DIM = _ENT_DIM * _EXPANSION
_BATCH = 16

_TILE_N = 2048
_NT = _OUT_DIM // _TILE_N


def _fused_kernel(x_ref, wt_ref, b_ref, o_ref):
    i = pl.program_id(0)
    acc = jax.lax.dot_general(
        x_ref[...], wt_ref[...],
        dimension_numbers=(((1,), (0,)), ((), ())),
        preferred_element_type=jnp.float32,
        precision=jax.lax.Precision.DEFAULT,
    )
    acc = jnp.maximum(acc + b_ref[...], 0.0)
    o_ref[:, pl.ds(i * _TILE_N, _TILE_N)] = acc

    @pl.when(i == _NT - 1)
    def _finalize():
        x = o_ref[...]
        xi = jax.lax.bitcast_convert_type(x, jnp.int32)

        def body(j, t):
            sh = 29 - 2 * j
            ch = t | (2 << sh)
            cl = t | (1 << sh)
            chl = ch | (1 << sh)
            nh = jnp.sum((xi >= ch).astype(jnp.int32), axis=1, keepdims=True)
            nl = jnp.sum((xi >= cl).astype(jnp.int32), axis=1, keepdims=True)
            nhl = jnp.sum((xi >= chl).astype(jnp.int32), axis=1, keepdims=True)
            return jnp.where(
                nh >= _K,
                jnp.where(nhl >= _K, chl, ch),
                jnp.where(nl >= _K, cl, t),
            )

        t = jax.lax.fori_loop(0, 15, body, jnp.zeros((_BATCH, 1), jnp.int32))
        c0 = t | 1
        n0 = jnp.sum((xi >= c0).astype(jnp.int32), axis=1, keepdims=True)
        t = jnp.where(n0 >= _K, c0, t)
        o_ref[...] = jnp.where(xi >= t, x, 0.0)


@jax.jit
def kernel(ent_output, W, b):
    b2 = b.reshape(1, _OUT_DIM)
    WT = W.T
    return pl.pallas_call(
        _fused_kernel,
        grid=(_NT,),
        in_specs=[
            pl.BlockSpec((_BATCH, _IN_DIM), lambda i: (0, 0)),
            pl.BlockSpec((_IN_DIM, _TILE_N), lambda i: (0, i)),
            pl.BlockSpec((1, _TILE_N), lambda i: (0, i)),
        ],
        out_specs=pl.BlockSpec((_BATCH, _OUT_DIM), lambda i: (0, 0)),
        out_shape=jax.ShapeDtypeStruct((_BATCH, _OUT_DIM), jnp.float32),
    )(ent_output, WT, b2)


# speedup vs baseline: 1.0013x; 1.0013x over previous
import jax
import jax.numpy as jnp
from jax.experimental import pallas as pl

_ENT_DIM = 2048
_EXPANSION = 16
_K = 512
_IN_DIM = _ENT_DIM + 1
_OUT_DIM = _ENT_DIM * _EXPANSION
_BATCH = 16

_TILE_N = 1024
_NT = _OUT_DIM // _TILE_N


def _fused_kernel(x_ref, wt_ref, b_ref, o_ref):
    i = pl.program_id(0)
    acc = jax.lax.dot_general(
        x_ref[...], wt_ref[...],
        dimension_numbers=(((1,), (0,)), ((), ())),
        preferred_element_type=jnp.float32,
        precision=jax.lax.Precision.DEFAULT,
    )
    acc = jnp.maximum(acc + b_ref[...], 0.0)
    o_ref[:, pl.ds(i * _TILE_N, _TILE_N)] = acc

    @pl.when(i == _NT - 1)
    def _finalize():
        x = o_ref[...]
        xi = jax.lax.bitcast_convert_type(x, jnp.int32)

        def body(j, t):
            cand = t | (1 << (30 - j))
            cnt = jnp.sum((xi >= cand).astype(jnp.int32), axis=1, keepdims=True)
            return jnp.where(cnt >= _K, cand, t)

        t = jax.lax.fori_loop(0, 31, body, jnp.zeros((_BATCH, 1), jnp.int32))
        o_ref[...] = jnp.where(xi >= t, x, 0.0)


@jax.jit
def kernel(ent_output, W, b):
    b2 = b.reshape(1, _OUT_DIM)
    WT = W.T
    return pl.pallas_call(
        _fused_kernel,
        grid=(_NT,),
        in_specs=[
            pl.BlockSpec((_BATCH, _IN_DIM), lambda i: (0, 0)),
            pl.BlockSpec((_IN_DIM, _TILE_N), lambda i: (0, i)),
            pl.BlockSpec((1, _TILE_N), lambda i: (0, i)),
        ],
        out_specs=pl.BlockSpec((_BATCH, _OUT_DIM), lambda i: (0, 0)),
        out_shape=jax.ShapeDtypeStruct((_BATCH, _OUT_DIM), jnp.float32),
    )(ent_output, WT, b2)


# speedup vs baseline: 1.0064x; 1.0050x over previous
import jax
import jax.numpy as jnp
from jax.experimental import pallas as pl

_ENT_DIM = 2048
_EXPANSION = 16
_K = 512
_IN_DIM = _ENT_DIM + 1
_OUT_DIM = _ENT_DIM * _EXPANSION
_BATCH = 16

_TILE_N = 2048
_NT = _OUT_DIM // _TILE_N


def _fused_kernel(x_ref, wt_ref, b_ref, o_ref):
    i = pl.program_id(0)
    acc = jax.lax.dot_general(
        x_ref[...], wt_ref[...],
        dimension_numbers=(((1,), (0,)), ((), ())),
        preferred_element_type=jnp.float32,
        precision=jax.lax.Precision.DEFAULT,
    )
    acc = jnp.maximum(acc + b_ref[...], 0.0)
    o_ref[:, pl.ds(i * _TILE_N, _TILE_N)] = acc

    @pl.when(i == _NT - 1)
    def _finalize():
        x = o_ref[...]
        xi = jax.lax.bitcast_convert_type(x, jnp.int32)

        def body(j, t):
            cand = t | (1 << (30 - j))
            cnt = jnp.sum((xi >= cand).astype(jnp.int32), axis=1, keepdims=True)
            return jnp.where(cnt >= _K, cand, t)

        t = jax.lax.fori_loop(0, 31, body, jnp.zeros((_BATCH, 1), jnp.int32))
        o_ref[...] = jnp.where(xi >= t, x, 0.0)


@jax.jit
def kernel(ent_output, W, b):
    b2 = b.reshape(1, _OUT_DIM)
    WT = W.T
    return pl.pallas_call(
        _fused_kernel,
        grid=(_NT,),
        in_specs=[
            pl.BlockSpec((_BATCH, _IN_DIM), lambda i: (0, 0)),
            pl.BlockSpec((_IN_DIM, _TILE_N), lambda i: (0, i)),
            pl.BlockSpec((1, _TILE_N), lambda i: (0, i)),
        ],
        out_specs=pl.BlockSpec((_BATCH, _OUT_DIM), lambda i: (0, 0)),
        out_shape=jax.ShapeDtypeStruct((_BATCH, _OUT_DIM), jnp.float32),
    )(ent_output, WT, b2)
